# baseline (speedup 1.0000x reference)
import functools

import jax
import jax.numpy as jnp
from jax import lax
from jax.experimental import pallas as pl
from jax.experimental.pallas import tpu as pltpu
from jax.experimental.pallas import tpu_sc as plsc

VOCAB = 100000
EMBED = 128
NPOS = 2048
B = 4
S = 2048

NC = 2
NS = 16
NW = NC * NS
NROWS = B * S
ROWS_PER_W = NROWS // NW
WPB = S // ROWS_PER_W
LANES = 16

CS = (48, 48, 32, 32, 32, 32, 16, 16)
CO = tuple(sum(CS[:j]) for j in range(len(CS)))
NCHUNK = len(CS)
HALF = ROWS_PER_W // 2
assert sum(CS) == ROWS_PER_W

_mesh = plsc.VectorSubcoreMesh(
    core_axis_name="c", subcore_axis_name="s", num_cores=NC, num_subcores=NS
)


@functools.partial(
    pl.kernel,
    out_type=jax.ShapeDtypeStruct((B, S, EMBED), jnp.float32),
    mesh=_mesh,
    scratch_types=[
        pltpu.VMEM((ROWS_PER_W,), jnp.int32),
        pltpu.VMEM((2, 48), jnp.int32),
        pltpu.VMEM((4, 32), jnp.int32),
        pltpu.VMEM((2, 16), jnp.int32),
        pltpu.VMEM((ROWS_PER_W, EMBED), jnp.float32),
        pltpu.VMEM_SHARED((NS * ROWS_PER_W, EMBED), jnp.float32),
        pltpu.SemaphoreType.DMA,
        pltpu.SemaphoreType.DMA,
        pltpu.SemaphoreType.DMA,
        pltpu.SemaphoreType.DMA,
        pltpu.SemaphoreType.DMA,
        pltpu.SemaphoreType.DMA,
        pltpu.SemaphoreType.DMA,
        pltpu.SemaphoreType.DMA,
        pltpu.SemaphoreType.DMA,
        pltpu.SemaphoreType.DMA,
        pltpu.SemaphoreType.DMA,
    ],
)
def _embed_kernel(x_hbm, tok_hbm, pos_hbm, out_hbm, idx_v, ids48, ids32,
                  ids16, tok_v, acc_sh, sem_in, sem_in2, sem0, sem1, sem2,
                  sem3, sem4, sem5, sem6, sem7, sem_out):
    sems = [sem0, sem1, sem2, sem3, sem4, sem5, sem6, sem7]
    cid = lax.axis_index("c")
    sid = lax.axis_index("s")
    wid = sid * NC + cid
    b = wid // WPB
    s0 = lax.rem(wid, WPB) * ROWS_PER_W
    spbase = sid * ROWS_PER_W

    idx_cp0 = pltpu.async_copy(
        x_hbm.at[b, pl.ds(s0, HALF)], idx_v.at[pl.ds(0, HALF)], sem_in
    )
    idx_cp1 = pltpu.async_copy(
        x_hbm.at[b, pl.ds(s0 + HALF, HALF)], idx_v.at[pl.ds(HALF, HALF)], sem_in2
    )
    pos_cps = [
        pltpu.async_copy(
            pos_hbm.at[pl.ds(s0 + CO[c], CS[c])],
            acc_sh.at[pl.ds(spbase + CO[c], CS[c])],
            sems[c],
        )
        for c in range(NCHUNK)
    ]

    lane = lax.iota(jnp.int32, 16)
    ids_rows = []
    counts = {}
    for c in range(NCHUNK):
        ref = {48: ids48, 32: ids32, 16: ids16}[CS[c]]
        row = counts.get(CS[c], 0)
        counts[CS[c]] = row + 1
        for k in range(CS[c] // LANES):
            ref[row, pl.ds(k * LANES, LANES)] = lane + (
                spbase + CO[c] + k * LANES
            )
        ids_rows.append(ref.at[row])

    first_half = [c for c in range(NCHUNK) if CO[c] + CS[c] <= HALF]
    second_half = [c for c in range(NCHUNK) if CO[c] + CS[c] > HALF]
    g_cps = [None] * NCHUNK
    idx_cp0.wait()
    for c in first_half:
        g_cps[c] = pltpu.async_copy(
            tok_hbm.at[idx_v.at[pl.ds(CO[c], CS[c])]],
            tok_v.at[pl.ds(CO[c], CS[c])],
            sems[c],
        )
    idx_cp1.wait()
    for c in second_half:
        g_cps[c] = pltpu.async_copy(
            tok_hbm.at[idx_v.at[pl.ds(CO[c], CS[c])]],
            tok_v.at[pl.ds(CO[c], CS[c])],
            sems[c],
        )

    sa_cps = []
    out_cps = []
    for c in range(NCHUNK):
        pos_cps[c].wait()
        g_cps[c].wait()
        sa_cps.append(
            pltpu.async_copy(
                tok_v.at[pl.ds(CO[c], CS[c])],
                acc_sh.at[ids_rows[c]],
                sems[c],
                add=True,
            )
        )
        if c >= 1:
            sa_cps[c - 1].wait()
            out_cps.append(
                pltpu.async_copy(
                    acc_sh.at[pl.ds(spbase + CO[c - 1], CS[c - 1])],
                    out_hbm.at[b, pl.ds(s0 + CO[c - 1], CS[c - 1])],
                    sem_out,
                )
            )
    c = NCHUNK - 1
    sa_cps[c].wait()
    out_cps.append(
        pltpu.async_copy(
            acc_sh.at[pl.ds(spbase + CO[c], CS[c])],
            out_hbm.at[b, pl.ds(s0 + CO[c], CS[c])],
            sem_out,
        )
    )
    for cp in out_cps:
        cp.wait()


def kernel(x, tok_table, pos_table):
    return _embed_kernel(x, tok_table, pos_table)
